# Initial kernel scaffold; baseline (speedup 1.0000x reference)
import jax
import jax.numpy as jnp
from jax.experimental import pallas as pl


def kernel(x, edge_index, batch, Wl0, bl0, Wr0, Wl1, bl1, Wr1, Wl2, bl2, Wr2, Wl3, bl3, Wr3, Wl4, bl4, Wr4, Wl5, bl5, Wr5, t):
    raise NotImplementedError("write your pallas kernel here")


# speedup vs baseline: 5.3740x; 5.3740x over previous
import functools

import jax
import jax.numpy as jnp
from jax import lax
from jax.experimental import pallas as pl
from jax.experimental.pallas import tpu as pltpu
from jax.experimental.pallas import tpu_sc as plsc

N = 10000
E = 320000
F = 128
G = 100

NC = 2
NS = 16
CHUNK = 128
NCHUNKS = E // CHUNK
SUB_ROWS = 632
SUB_ROWS_LAST = N - (NS - 1) * SUB_ROWS

_SC_MESH = plsc.VectorSubcoreMesh(core_axis_name="c", subcore_axis_name="s",
                                  num_cores=NC, num_subcores=NS)


def _fill_zeros(buf, nrows, ncols):
    zv = jnp.zeros((16,), jnp.float32)

    def body(i, _):
        for j in range(ncols // 16):
            buf[i, pl.ds(j * 16, 16)] = zv
        return 0

    lax.fori_loop(0, nrows, body, 0)


def _zero_range(zbuf, acc, base, s):
    for k in range(4):
        pltpu.sync_copy(zbuf, acc.at[pl.ds(base + k * 128, 128)])

    @pl.when(s < NS - 1)
    def _():
        pltpu.sync_copy(zbuf.at[pl.ds(0, SUB_ROWS - 512)],
                        acc.at[pl.ds(base + 512, SUB_ROWS - 512)])

    @pl.when(s == NS - 1)
    def _():
        pltpu.sync_copy(zbuf.at[pl.ds(0, SUB_ROWS_LAST - 512)],
                        acc.at[pl.ds(base + 512, SUB_ROWS_LAST - 512)])


def _sliced_copy2(src, dst, base, s):
    @pl.when(s < NS - 1)
    def _():
        pltpu.sync_copy(src.at[pl.ds(base, SUB_ROWS)],
                        dst.at[pl.ds(base, SUB_ROWS)])

    @pl.when(s == NS - 1)
    def _():
        pltpu.sync_copy(src.at[pl.ds(base, SUB_ROWS_LAST)],
                        dst.at[pl.ds(base, SUB_ROWS_LAST)])


def _agg_body(y_hbm, src_hbm, dst_hbm, out0, out1, acc, sidx, didx, rows,
              zbuf, sem):
    c = lax.axis_index("c")
    s = lax.axis_index("s")
    wid = s * NC + c

    _fill_zeros(zbuf, 128, F)
    base = pl.multiple_of(s * SUB_ROWS, 8)
    _zero_range(zbuf, acc, base, s)
    plsc.subcore_barrier()

    nchunks = 78 + jnp.where(wid < NCHUNKS - 32 * 78, 1, 0)

    def chunk_body(i, _):
        t = wid + i * 32
        off = t * CHUNK
        pltpu.sync_copy(src_hbm.at[pl.ds(off, CHUNK)], sidx.at[0])
        pltpu.sync_copy(dst_hbm.at[pl.ds(off, CHUNK)], didx.at[0])
        pltpu.async_copy(y_hbm.at[sidx.at[0]], rows, sem).wait()
        pltpu.sync_copy(rows, acc.at[didx.at[0]], add=True)
        return 0

    lax.fori_loop(0, nchunks, chunk_body, 0)
    plsc.subcore_barrier()

    @pl.when(c == 0)
    def _():
        _sliced_copy2(acc, out0, base, s)

    @pl.when(c == 1)
    def _():
        _sliced_copy2(acc, out1, base, s)


_sc_aggregate = pl.kernel(
    _agg_body,
    out_type=(jax.ShapeDtypeStruct((N, F), jnp.float32),
              jax.ShapeDtypeStruct((N, F), jnp.float32)),
    mesh=_SC_MESH,
    scratch_types=[
        pltpu.VMEM_SHARED((N, F), jnp.float32),
        pltpu.VMEM((1, CHUNK), jnp.int32),
        pltpu.VMEM((1, CHUNK), jnp.int32),
        pltpu.VMEM((CHUNK, F), jnp.float32),
        pltpu.VMEM((128, F), jnp.float32),
        pltpu.SemaphoreType.DMA,
    ],
)


def _cnt_body(dst_hbm, out0, out1, acc, didx, ones, zbuf, sem):
    del sem
    c = lax.axis_index("c")
    s = lax.axis_index("s")
    wid = s * NC + c

    _fill_zeros(zbuf, 128, F)
    base = pl.multiple_of(s * SUB_ROWS, 8)
    _zero_range(zbuf, acc, base, s)

    ov = jnp.ones((16,), jnp.float32)

    def fill_ones(i, _):
        for j in range(F // 16):
            ones[i, pl.ds(j * 16, 16)] = ov
        return 0

    lax.fori_loop(0, CHUNK, fill_ones, 0)
    plsc.subcore_barrier()

    nchunks = 78 + jnp.where(wid < NCHUNKS - 32 * 78, 1, 0)

    def chunk_body(i, _):
        t = wid + i * 32
        pltpu.sync_copy(dst_hbm.at[pl.ds(t * CHUNK, CHUNK)], didx.at[0])
        pltpu.sync_copy(ones, acc.at[didx.at[0]], add=True)
        return 0

    lax.fori_loop(0, nchunks, chunk_body, 0)
    plsc.subcore_barrier()

    @pl.when(c == 0)
    def _():
        _sliced_copy2(acc, out0, base, s)

    @pl.when(c == 1)
    def _():
        _sliced_copy2(acc, out1, base, s)


_sc_counts = pl.kernel(
    _cnt_body,
    out_type=(jax.ShapeDtypeStruct((N, F), jnp.float32),
              jax.ShapeDtypeStruct((N, F), jnp.float32)),
    mesh=_SC_MESH,
    scratch_types=[
        pltpu.VMEM_SHARED((N, F), jnp.float32),
        pltpu.VMEM((1, CHUNK), jnp.int32),
        pltpu.VMEM((CHUNK, F), jnp.float32),
        pltpu.VMEM((128, F), jnp.float32),
        pltpu.SemaphoreType.DMA,
    ],
)


BLK = 2000


def _mm2(xb, wl, wr, bl):
    y = lax.dot_general(xb, wl, (((1,), (1,)), ((), ())),
                        preferred_element_type=jnp.float32)
    z = lax.dot_general(xb, wr, (((1,), (1,)), ((), ())),
                        preferred_element_type=jnp.float32) + bl[0][None, :]
    return y, z


def _tc_first_body(x_ref, wl_ref, wr_ref, bl_ref, y_ref, z_ref):
    y, z = _mm2(x_ref[...], wl_ref[...], wr_ref[...], bl_ref)
    y_ref[...] = y
    z_ref[...] = z


def _tc_first(x, wl, wr, bl):
    return pl.pallas_call(
        _tc_first_body,
        grid=(N // BLK,),
        in_specs=[
            pl.BlockSpec((BLK, F), lambda i: (i, 0)),
            pl.BlockSpec((F, F), lambda i: (0, 0)),
            pl.BlockSpec((F, F), lambda i: (0, 0)),
            pl.BlockSpec((1, F), lambda i: (0, 0)),
        ],
        out_specs=[
            pl.BlockSpec((BLK, F), lambda i: (i, 0)),
            pl.BlockSpec((BLK, F), lambda i: (i, 0)),
        ],
        out_shape=[jax.ShapeDtypeStruct((N, F), jnp.float32),
                   jax.ShapeDtypeStruct((N, F), jnp.float32)],
    )(x, wl, wr, bl.reshape(1, F))


def _combine(x_ref, p0_ref, p1_ref, inv_ref, zp_ref):
    mean = (p0_ref[...] + p1_ref[...]) * inv_ref[...]
    return jnp.maximum(mean + zp_ref[...], 0.0) + x_ref[...]


def _tc_mid_body(x_ref, p0_ref, p1_ref, inv_ref, zp_ref, wl_ref, wr_ref,
                 bl_ref, xn_ref, y_ref, z_ref):
    xn = _combine(x_ref, p0_ref, p1_ref, inv_ref, zp_ref)
    xn_ref[...] = xn
    y, z = _mm2(xn, wl_ref[...], wr_ref[...], bl_ref)
    y_ref[...] = y
    z_ref[...] = z


def _tc_mid(x, p0, p1, invb, zp, wl, wr, bl):
    blkspec = pl.BlockSpec((BLK, F), lambda i: (i, 0))
    wspec = pl.BlockSpec((F, F), lambda i: (0, 0))
    return pl.pallas_call(
        _tc_mid_body,
        grid=(N // BLK,),
        in_specs=[blkspec, blkspec, blkspec, blkspec, blkspec, wspec, wspec,
                  pl.BlockSpec((1, F), lambda i: (0, 0))],
        out_specs=[blkspec, blkspec, blkspec],
        out_shape=[jax.ShapeDtypeStruct((N, F), jnp.float32)] * 3,
    )(x, p0, p1, invb, zp, wl, wr, bl.reshape(1, F))


def _tc_last_body(x_ref, p0_ref, p1_ref, inv_ref, zp_ref, xn_ref):
    xn_ref[...] = _combine(x_ref, p0_ref, p1_ref, inv_ref, zp_ref)


def _tc_last(x, p0, p1, invb, zp):
    blkspec = pl.BlockSpec((BLK, F), lambda i: (i, 0))
    return pl.pallas_call(
        _tc_last_body,
        grid=(N // BLK,),
        in_specs=[blkspec] * 5,
        out_specs=blkspec,
        out_shape=jax.ShapeDtypeStruct((N, F), jnp.float32),
    )(x, p0, p1, invb, zp)


SBLK = 1000
GPAD = 104


def _softmax_body(t_ref, x_ref, b_ref, out_ref, m_ref, den_ref, num_ref):
    p = pl.program_id(0)
    blk = pl.program_id(1)
    t = t_ref[0, 0]
    xb = x_ref[...]
    bb = b_ref[...]
    g_lo = jnp.min(bb)
    g_hi = jnp.max(bb)

    @pl.when(jnp.logical_and(p == 0, blk == 0))
    def _():
        m_ref[...] = jnp.full((GPAD, F), -3.0e38, jnp.float32)
        den_ref[...] = jnp.zeros((GPAD, F), jnp.float32)
        num_ref[...] = jnp.zeros((GPAD, F), jnp.float32)

    logits = xb * t

    @pl.when(p == 0)
    def _():
        def body(g, _):
            sel = jnp.where(bb == g, logits, -3.0e38)
            m = jnp.max(sel, axis=0, keepdims=True)
            m_ref[pl.ds(g, 1), :] = jnp.maximum(m_ref[pl.ds(g, 1), :], m)
            return 0

        lax.fori_loop(g_lo, g_hi + 1, body, 0)

    @pl.when(p == 1)
    def _():
        def body(g, _):
            mask = bb == g
            mg = m_ref[pl.ds(g, 1), :]
            e = jnp.where(mask, jnp.exp(logits - mg), 0.0)
            den_ref[pl.ds(g, 1), :] += jnp.sum(e, axis=0, keepdims=True)
            num_ref[pl.ds(g, 1), :] += jnp.sum(xb * e, axis=0, keepdims=True)
            return 0

        lax.fori_loop(g_lo, g_hi + 1, body, 0)

    @pl.when(jnp.logical_and(p == 1, blk == pl.num_programs(1) - 1))
    def _():
        den = den_ref[...]
        out_ref[...] = jnp.where(den > 0.0, num_ref[...] / den, 0.0)


def _tc_softmax(x, batchb, t):
    return pl.pallas_call(
        _softmax_body,
        grid=(2, N // SBLK),
        in_specs=[
            pl.BlockSpec(memory_space=pltpu.SMEM),
            pl.BlockSpec((SBLK, F), lambda p, b: (b, 0)),
            pl.BlockSpec((SBLK, F), lambda p, b: (b, 0)),
        ],
        out_specs=pl.BlockSpec((GPAD, F), lambda p, b: (0, 0)),
        out_shape=jax.ShapeDtypeStruct((GPAD, F), jnp.float32),
        scratch_shapes=[
            pltpu.VMEM((GPAD, F), jnp.float32),
            pltpu.VMEM((GPAD, F), jnp.float32),
            pltpu.VMEM((GPAD, F), jnp.float32),
        ],
        compiler_params=pltpu.CompilerParams(
            dimension_semantics=("arbitrary", "arbitrary")),
    )(t.reshape(1, 1), x, batchb)


def kernel(x, edge_index, batch,
           Wl0, bl0, Wr0, Wl1, bl1, Wr1, Wl2, bl2, Wr2,
           Wl3, bl3, Wr3, Wl4, bl4, Wr4, Wl5, bl5, Wr5, t):
    src = edge_index[0]
    dst = edge_index[1]

    c0, c1 = _sc_counts(dst)
    cnt = c0[:, 0] + c1[:, 0]
    invb = jnp.broadcast_to((1.0 / jnp.maximum(cnt, 1.0))[:, None], (N, F))
    batchb = jnp.broadcast_to(batch[:, None], (N, F))

    params = [(Wl0, bl0, Wr0), (Wl1, bl1, Wr1), (Wl2, bl2, Wr2),
              (Wl3, bl3, Wr3), (Wl4, bl4, Wr4), (Wl5, bl5, Wr5)]

    y, z = _tc_first(x, Wl0, Wr0, bl0)
    for i in range(6):
        p0, p1 = _sc_aggregate(y, src, dst)
        if i < 5:
            wl, bl, wr = params[i + 1]
            x, y, z = _tc_mid(x, p0, p1, invb, z, wl, wr, bl)
        else:
            x = _tc_last(x, p0, p1, invb, z)

    return _tc_softmax(x, batchb, t)[:G]


# speedup vs baseline: 8.7433x; 1.6270x over previous
import functools

import jax
import jax.numpy as jnp
from jax import lax
from jax.experimental import pallas as pl
from jax.experimental.pallas import tpu as pltpu
from jax.experimental.pallas import tpu_sc as plsc

N = 10000
E = 320000
F = 128
G = 100

NC = 2
NS = 16
NW = NC * NS
CHUNK = 128
WCHUNKS = 80
EPAD = NW * WCHUNKS * CHUNK
NROWS_PAD = 64
NP = N + NROWS_PAD
IBLK = 32
SUB_ROWS = 632
SUB_ROWS_LAST = NP - (NS - 1) * SUB_ROWS

_SC_MESH = plsc.VectorSubcoreMesh(core_axis_name="c", subcore_axis_name="s",
                                  num_cores=NC, num_subcores=NS)


def _fill_const(buf, nrows, ncols, val):
    vv = jnp.full((16,), val, jnp.float32)

    def body(i, _):
        for j in range(ncols // 16):
            buf[i, pl.ds(j * 16, 16)] = vv
        return 0

    lax.fori_loop(0, nrows, body, 0)


def _zero_range(zbuf, acc, base, s):
    for k in range(4):
        pltpu.sync_copy(zbuf, acc.at[pl.ds(base + k * 128, 128)])

    @pl.when(s < NS - 1)
    def _():
        pltpu.sync_copy(zbuf.at[pl.ds(0, SUB_ROWS - 512)],
                        acc.at[pl.ds(base + 512, SUB_ROWS - 512)])

    @pl.when(s == NS - 1)
    def _():
        pltpu.sync_copy(zbuf.at[pl.ds(0, SUB_ROWS_LAST - 512)],
                        acc.at[pl.ds(base + 512, SUB_ROWS_LAST - 512)])


def _sliced_copy2(src, dst, base, s):
    @pl.when(s < NS - 1)
    def _():
        pltpu.sync_copy(src.at[pl.ds(base, SUB_ROWS)],
                        dst.at[pl.ds(base, SUB_ROWS)])

    @pl.when(s == NS - 1)
    def _():
        pltpu.sync_copy(src.at[pl.ds(base, SUB_ROWS_LAST)],
                        dst.at[pl.ds(base, SUB_ROWS_LAST)])


def _agg_body(y_hbm, src_hbm, dst_hbm, out0, out1, acc,
              sidx0, sidx1, didx0, didx1, rows0, rows1,
              isem0, isem1, gsem0, gsem1):
    c = lax.axis_index("c")
    s = lax.axis_index("s")
    wid = s * NC + c
    sidx = (sidx0, sidx1)
    didx = (didx0, didx1)
    rows = (rows0, rows1)
    isem = (isem0, isem1)
    gsem = (gsem0, gsem1)

    _fill_const(rows0, 128, F, 0.0)
    base = pl.multiple_of(s * SUB_ROWS, 8)
    _zero_range(rows0, acc, base, s)
    plsc.subcore_barrier()

    wbase = wid * WCHUNKS

    def _off(k):
        return pl.multiple_of((wbase + k) * CHUNK, 8)

    def idx_issue(p, k):
        pltpu.async_copy(src_hbm.at[pl.ds(_off(k), CHUNK)], sidx[p].at[0],
                         isem[p])
        pltpu.async_copy(dst_hbm.at[pl.ds(_off(k), CHUNK)], didx[p].at[0],
                         isem[p])

    def idx_wait(p, k):
        pltpu.make_async_copy(src_hbm.at[pl.ds(_off(k), CHUNK)],
                              sidx[p].at[0], isem[p]).wait()
        pltpu.make_async_copy(dst_hbm.at[pl.ds(_off(k), CHUNK)],
                              didx[p].at[0], isem[p]).wait()

    def gather_issue(p):
        pltpu.async_copy(y_hbm.at[sidx[p].at[0]], rows[p], gsem[p])

    def gather_wait(p):
        pltpu.make_async_copy(y_hbm.at[sidx[p].at[0]], rows[p],
                              gsem[p]).wait()

    idx_issue(0, 0)
    idx_issue(1, 1)
    idx_wait(0, 0)
    gather_issue(0)

    def pair(j, _):
        for p in (0, 1):
            r = j * 2 + p
            np_ = 1 - p
            gather_wait(p)
            if p == 0:
                idx_wait(np_, r + 1)
                gather_issue(np_)
            else:
                @pl.when(j < WCHUNKS // 2 - 1)
                def _():
                    idx_wait(np_, r + 1)
                    gather_issue(np_)
            pltpu.sync_copy(rows[p], acc.at[didx[p].at[0]], add=True)

            @pl.when(j < WCHUNKS // 2 - 1)
            def _():
                idx_issue(p, r + 2)
        return 0

    lax.fori_loop(0, WCHUNKS // 2, pair, 0)
    plsc.subcore_barrier()

    @pl.when(c == 0)
    def _():
        _sliced_copy2(acc, out0, base, s)

    @pl.when(c == 1)
    def _():
        _sliced_copy2(acc, out1, base, s)


_sc_aggregate = pl.kernel(
    _agg_body,
    out_type=(jax.ShapeDtypeStruct((NP, F), jnp.float32),
              jax.ShapeDtypeStruct((NP, F), jnp.float32)),
    mesh=_SC_MESH,
    scratch_types=[
        pltpu.VMEM_SHARED((NP, F), jnp.float32),
        pltpu.VMEM((1, CHUNK), jnp.int32),
        pltpu.VMEM((1, CHUNK), jnp.int32),
        pltpu.VMEM((1, CHUNK), jnp.int32),
        pltpu.VMEM((1, CHUNK), jnp.int32),
        pltpu.VMEM((CHUNK, F), jnp.float32),
        pltpu.VMEM((CHUNK, F), jnp.float32),
        pltpu.SemaphoreType.DMA,
        pltpu.SemaphoreType.DMA,
        pltpu.SemaphoreType.DMA,
        pltpu.SemaphoreType.DMA,
    ],
)


def _cnt_body(dst_hbm, out0, out1, acc, didx0, didx1, ones, isem0, isem1):
    c = lax.axis_index("c")
    s = lax.axis_index("s")
    wid = s * NC + c
    didx = (didx0, didx1)
    isem = (isem0, isem1)

    _fill_const(ones, 128, F, 0.0)
    base = pl.multiple_of(s * SUB_ROWS, 8)
    _zero_range(ones, acc, base, s)
    _fill_const(ones, 128, F, 1.0)
    plsc.subcore_barrier()

    wbase = wid * WCHUNKS

    def _off(k):
        return pl.multiple_of((wbase + k) * CHUNK, 8)

    def idx_issue(p, k):
        pltpu.async_copy(dst_hbm.at[pl.ds(_off(k), CHUNK)], didx[p].at[0],
                         isem[p])

    def idx_wait(p, k):
        pltpu.make_async_copy(dst_hbm.at[pl.ds(_off(k), CHUNK)],
                              didx[p].at[0], isem[p]).wait()

    idx_issue(0, 0)
    idx_issue(1, 1)

    def pair(j, _):
        for p in (0, 1):
            r = j * 2 + p
            idx_wait(p, r)
            pltpu.sync_copy(ones, acc.at[didx[p].at[0]], add=True)

            @pl.when(j < WCHUNKS // 2 - 1)
            def _():
                idx_issue(p, r + 2)
        return 0

    lax.fori_loop(0, WCHUNKS // 2, pair, 0)
    plsc.subcore_barrier()

    @pl.when(c == 0)
    def _():
        _sliced_copy2(acc, out0, base, s)

    @pl.when(c == 1)
    def _():
        _sliced_copy2(acc, out1, base, s)


_sc_counts = pl.kernel(
    _cnt_body,
    out_type=(jax.ShapeDtypeStruct((NP, F), jnp.float32),
              jax.ShapeDtypeStruct((NP, F), jnp.float32)),
    mesh=_SC_MESH,
    scratch_types=[
        pltpu.VMEM_SHARED((NP, F), jnp.float32),
        pltpu.VMEM((1, CHUNK), jnp.int32),
        pltpu.VMEM((1, CHUNK), jnp.int32),
        pltpu.VMEM((CHUNK, F), jnp.float32),
        pltpu.SemaphoreType.DMA,
        pltpu.SemaphoreType.DMA,
    ],
)


BLK = 2000


def _mm2(xb, wl, wr, bl):
    y = lax.dot_general(xb, wl, (((1,), (1,)), ((), ())),
                        preferred_element_type=jnp.float32)
    z = lax.dot_general(xb, wr, (((1,), (1,)), ((), ())),
                        preferred_element_type=jnp.float32) + bl[0][None, :]
    return y, z


def _tc_first_body(x_ref, wl_ref, wr_ref, bl_ref, y_ref, z_ref):
    y, z = _mm2(x_ref[...], wl_ref[...], wr_ref[...], bl_ref)
    y_ref[...] = y
    z_ref[...] = z


def _tc_first(x, wl, wr, bl):
    return pl.pallas_call(
        _tc_first_body,
        grid=(N // BLK,),
        in_specs=[
            pl.BlockSpec((BLK, F), lambda i: (i, 0)),
            pl.BlockSpec((F, F), lambda i: (0, 0)),
            pl.BlockSpec((F, F), lambda i: (0, 0)),
            pl.BlockSpec((1, F), lambda i: (0, 0)),
        ],
        out_specs=[
            pl.BlockSpec((BLK, F), lambda i: (i, 0)),
            pl.BlockSpec((BLK, F), lambda i: (i, 0)),
        ],
        out_shape=[jax.ShapeDtypeStruct((N, F), jnp.float32),
                   jax.ShapeDtypeStruct((N, F), jnp.float32)],
    )(x, wl, wr, bl.reshape(1, F))


def _combine(x_ref, p0_ref, p1_ref, inv_ref, zp_ref):
    mean = (p0_ref[...] + p1_ref[...]) * inv_ref[...]
    return jnp.maximum(mean + zp_ref[...], 0.0) + x_ref[...]


def _tc_mid_body(x_ref, p0_ref, p1_ref, inv_ref, zp_ref, wl_ref, wr_ref,
                 bl_ref, xn_ref, y_ref, z_ref):
    xn = _combine(x_ref, p0_ref, p1_ref, inv_ref, zp_ref)
    xn_ref[...] = xn
    y, z = _mm2(xn, wl_ref[...], wr_ref[...], bl_ref)
    y_ref[...] = y
    z_ref[...] = z


def _tc_mid(x, p0, p1, invb, zp, wl, wr, bl):
    blkspec = pl.BlockSpec((BLK, F), lambda i: (i, 0))
    wspec = pl.BlockSpec((F, F), lambda i: (0, 0))
    return pl.pallas_call(
        _tc_mid_body,
        grid=(N // BLK,),
        in_specs=[blkspec, blkspec, blkspec, blkspec, blkspec, wspec, wspec,
                  pl.BlockSpec((1, F), lambda i: (0, 0))],
        out_specs=[blkspec, blkspec, blkspec],
        out_shape=[jax.ShapeDtypeStruct((N, F), jnp.float32)] * 3,
    )(x, p0, p1, invb, zp, wl, wr, bl.reshape(1, F))


def _tc_last_body(x_ref, p0_ref, p1_ref, inv_ref, zp_ref, xn_ref):
    xn_ref[...] = _combine(x_ref, p0_ref, p1_ref, inv_ref, zp_ref)


def _tc_last(x, p0, p1, invb, zp):
    blkspec = pl.BlockSpec((BLK, F), lambda i: (i, 0))
    return pl.pallas_call(
        _tc_last_body,
        grid=(N // BLK,),
        in_specs=[blkspec] * 5,
        out_specs=blkspec,
        out_shape=jax.ShapeDtypeStruct((N, F), jnp.float32),
    )(x, p0, p1, invb, zp)


SBLK = 1000
GPAD = 104


def _softmax_body(t_ref, x_ref, b_ref, out_ref, m_ref, den_ref, num_ref):
    p = pl.program_id(0)
    blk = pl.program_id(1)
    t = t_ref[0, 0]
    xb = x_ref[...]
    bb = b_ref[...]
    g_lo = jnp.min(bb)
    g_hi = jnp.max(bb)

    @pl.when(jnp.logical_and(p == 0, blk == 0))
    def _():
        m_ref[...] = jnp.full((GPAD, F), -3.0e38, jnp.float32)
        den_ref[...] = jnp.zeros((GPAD, F), jnp.float32)
        num_ref[...] = jnp.zeros((GPAD, F), jnp.float32)

    logits = xb * t

    @pl.when(p == 0)
    def _():
        def body(g, _):
            sel = jnp.where(bb == g, logits, -3.0e38)
            m = jnp.max(sel, axis=0, keepdims=True)
            m_ref[pl.ds(g, 1), :] = jnp.maximum(m_ref[pl.ds(g, 1), :], m)
            return 0

        lax.fori_loop(g_lo, g_hi + 1, body, 0)

    @pl.when(p == 1)
    def _():
        def body(g, _):
            mask = bb == g
            mg = m_ref[pl.ds(g, 1), :]
            e = jnp.where(mask, jnp.exp(logits - mg), 0.0)
            den_ref[pl.ds(g, 1), :] += jnp.sum(e, axis=0, keepdims=True)
            num_ref[pl.ds(g, 1), :] += jnp.sum(xb * e, axis=0, keepdims=True)
            return 0

        lax.fori_loop(g_lo, g_hi + 1, body, 0)

    @pl.when(jnp.logical_and(p == 1, blk == pl.num_programs(1) - 1))
    def _():
        den = den_ref[...]
        out_ref[...] = jnp.where(den > 0.0, num_ref[...] / den, 0.0)


def _tc_softmax(x, batchb, t):
    return pl.pallas_call(
        _softmax_body,
        grid=(2, N // SBLK),
        in_specs=[
            pl.BlockSpec(memory_space=pltpu.SMEM),
            pl.BlockSpec((SBLK, F), lambda p, b: (b, 0)),
            pl.BlockSpec((SBLK, F), lambda p, b: (b, 0)),
        ],
        out_specs=pl.BlockSpec((GPAD, F), lambda p, b: (0, 0)),
        out_shape=jax.ShapeDtypeStruct((GPAD, F), jnp.float32),
        scratch_shapes=[
            pltpu.VMEM((GPAD, F), jnp.float32),
            pltpu.VMEM((GPAD, F), jnp.float32),
            pltpu.VMEM((GPAD, F), jnp.float32),
        ],
        compiler_params=pltpu.CompilerParams(
            dimension_semantics=("arbitrary", "arbitrary")),
    )(t.reshape(1, 1), x, batchb)


def _pad_edges(src, dst):
    npad = EPAD - E
    pad_i = jnp.arange(npad, dtype=jnp.int32)
    src_p = jnp.concatenate([src, (pad_i * 37) % N])
    dst_p = jnp.concatenate([dst, N + (pad_i % NROWS_PAD)])
    return src_p, dst_p


def kernel(x, edge_index, batch,
           Wl0, bl0, Wr0, Wl1, bl1, Wr1, Wl2, bl2, Wr2,
           Wl3, bl3, Wr3, Wl4, bl4, Wr4, Wl5, bl5, Wr5, t):
    src_p, dst_p = _pad_edges(edge_index[0], edge_index[1])

    c0, c1 = _sc_counts(dst_p)
    cnt = c0[:N, 0] + c1[:N, 0]
    invb = jnp.broadcast_to((1.0 / jnp.maximum(cnt, 1.0))[:, None], (N, F))
    batchb = jnp.broadcast_to(batch[:, None], (N, F))

    params = [(Wl0, bl0, Wr0), (Wl1, bl1, Wr1), (Wl2, bl2, Wr2),
              (Wl3, bl3, Wr3), (Wl4, bl4, Wr4), (Wl5, bl5, Wr5)]

    y, z = _tc_first(x, Wl0, Wr0, bl0)
    for i in range(6):
        p0, p1 = _sc_aggregate(y, src_p, dst_p)
        p0 = p0[:N]
        p1 = p1[:N]
        if i < 5:
            wl, bl, wr = params[i + 1]
            x, y, z = _tc_mid(x, p0, p1, invb, z, wl, wr, bl)
        else:
            x = _tc_last(x, p0, p1, invb, z)

    return _tc_softmax(x, batchb, t)[:G]


# speedup vs baseline: 9.1782x; 1.0497x over previous
import functools

import jax
import jax.numpy as jnp
from jax import lax
from jax.experimental import pallas as pl
from jax.experimental.pallas import tpu as pltpu
from jax.experimental.pallas import tpu_sc as plsc

N = 10000
E = 320000
F = 128
G = 100

NC = 2
NS = 16
NW = NC * NS
CHUNK = 128
WCHUNKS = 80
EPAD = NW * WCHUNKS * CHUNK
NROWS_PAD = 64
NP = N + NROWS_PAD
IBLK = 32
SUB_ROWS = 632
SUB_ROWS_LAST = NP - (NS - 1) * SUB_ROWS

_SC_MESH = plsc.VectorSubcoreMesh(core_axis_name="c", subcore_axis_name="s",
                                  num_cores=NC, num_subcores=NS)


def _fill_const(buf, nrows, ncols, val):
    vv = jnp.full((16,), val, jnp.float32)

    def body(i, _):
        for j in range(ncols // 16):
            buf[i, pl.ds(j * 16, 16)] = vv
        return 0

    lax.fori_loop(0, nrows, body, 0)


def _zero_range(zbuf, acc, base, s):
    for k in range(4):
        pltpu.sync_copy(zbuf, acc.at[pl.ds(base + k * 128, 128)])

    @pl.when(s < NS - 1)
    def _():
        pltpu.sync_copy(zbuf.at[pl.ds(0, SUB_ROWS - 512)],
                        acc.at[pl.ds(base + 512, SUB_ROWS - 512)])

    @pl.when(s == NS - 1)
    def _():
        pltpu.sync_copy(zbuf.at[pl.ds(0, SUB_ROWS_LAST - 512)],
                        acc.at[pl.ds(base + 512, SUB_ROWS_LAST - 512)])


def _sliced_copy2(src, dst, base, s):
    @pl.when(s < NS - 1)
    def _():
        pltpu.sync_copy(src.at[pl.ds(base, SUB_ROWS)],
                        dst.at[pl.ds(base, SUB_ROWS)])

    @pl.when(s == NS - 1)
    def _():
        pltpu.sync_copy(src.at[pl.ds(base, SUB_ROWS_LAST)],
                        dst.at[pl.ds(base, SUB_ROWS_LAST)])


def _agg_body(y_hbm, src_hbm, dst_hbm, out0, out1, acc,
              sidx0, sidx1, sidx2, sidx3, didx0, didx1, didx2, didx3,
              rows0, rows1, isem0, isem1, isem2, isem3,
              gsem0, gsem1, ssem0, ssem1):
    c = lax.axis_index("c")
    s = lax.axis_index("s")
    wid = s * NC + c
    sidx = (sidx0, sidx1, sidx2, sidx3)
    didx = (didx0, didx1, didx2, didx3)
    rows = (rows0, rows1)
    isem = (isem0, isem1, isem2, isem3)
    gsem = (gsem0, gsem1)
    ssem = (ssem0, ssem1)

    _fill_const(rows0, 128, F, 0.0)
    base = pl.multiple_of(s * SUB_ROWS, 8)
    _zero_range(rows0, acc, base, s)
    plsc.subcore_barrier()

    wbase = wid * WCHUNKS

    def _off(k):
        return pl.multiple_of((wbase + k) * CHUNK, 8)

    def idx_issue(q, k):
        pltpu.async_copy(src_hbm.at[pl.ds(_off(k), CHUNK)], sidx[q].at[0],
                         isem[q])
        pltpu.async_copy(dst_hbm.at[pl.ds(_off(k), CHUNK)], didx[q].at[0],
                         isem[q])

    def idx_wait(q, k):
        pltpu.make_async_copy(src_hbm.at[pl.ds(_off(k), CHUNK)],
                              sidx[q].at[0], isem[q]).wait()
        pltpu.make_async_copy(dst_hbm.at[pl.ds(_off(k), CHUNK)],
                              didx[q].at[0], isem[q]).wait()

    def gather_issue(b, q):
        pltpu.async_copy(y_hbm.at[sidx[q].at[0]], rows[b], gsem[b])

    def gather_wait(b, q):
        pltpu.make_async_copy(y_hbm.at[sidx[q].at[0]], rows[b],
                              gsem[b]).wait()

    def scatter_issue(b, q):
        pltpu.async_copy(rows[b], acc.at[didx[q].at[0]], ssem[b], add=True)

    def scatter_wait(b, q):
        pltpu.make_async_copy(rows[b], acc.at[didx[q].at[0]],
                              ssem[b]).wait()

    idx_issue(0, 0)
    idx_issue(1, 1)
    idx_wait(0, 0)
    gather_issue(0, 0)

    def quad(j, _):
        for u in range(4):
            r = j * 4 + u
            b = u % 2
            nb = 1 - b
            gather_wait(b, u)
            if u == 0:
                @pl.when(j > 0)
                def _():
                    scatter_wait(nb, 3)
            else:
                scatter_wait(nb, u - 1)

            if u < 3:
                idx_wait(u + 1, r + 1)
                gather_issue(nb, u + 1)
            else:
                @pl.when(j < WCHUNKS // 4 - 1)
                def _():
                    idx_wait(0, r + 1)
                    gather_issue(nb, 0)
            scatter_issue(b, u)

            @pl.when(jnp.logical_or(j < WCHUNKS // 4 - 1, u < 2))
            def _():
                idx_issue((u + 2) % 4, r + 2)
        return 0

    lax.fori_loop(0, WCHUNKS // 4, quad, 0)
    scatter_wait(1, 3)
    plsc.subcore_barrier()

    @pl.when(c == 0)
    def _():
        _sliced_copy2(acc, out0, base, s)

    @pl.when(c == 1)
    def _():
        _sliced_copy2(acc, out1, base, s)


_sc_aggregate = pl.kernel(
    _agg_body,
    out_type=(jax.ShapeDtypeStruct((NP, F), jnp.float32),
              jax.ShapeDtypeStruct((NP, F), jnp.float32)),
    mesh=_SC_MESH,
    scratch_types=(
        [pltpu.VMEM_SHARED((NP, F), jnp.float32)]
        + [pltpu.VMEM((1, CHUNK), jnp.int32)] * 8
        + [pltpu.VMEM((CHUNK, F), jnp.float32)] * 2
        + [pltpu.SemaphoreType.DMA] * 8
    ),
)


def _cnt_body(dst_hbm, out0, out1, acc, didx0, didx1, ones, isem0, isem1):
    c = lax.axis_index("c")
    s = lax.axis_index("s")
    wid = s * NC + c
    didx = (didx0, didx1)
    isem = (isem0, isem1)

    _fill_const(ones, 128, F, 0.0)
    base = pl.multiple_of(s * SUB_ROWS, 8)
    _zero_range(ones, acc, base, s)
    _fill_const(ones, 128, F, 1.0)
    plsc.subcore_barrier()

    wbase = wid * WCHUNKS

    def _off(k):
        return pl.multiple_of((wbase + k) * CHUNK, 8)

    def idx_issue(p, k):
        pltpu.async_copy(dst_hbm.at[pl.ds(_off(k), CHUNK)], didx[p].at[0],
                         isem[p])

    def idx_wait(p, k):
        pltpu.make_async_copy(dst_hbm.at[pl.ds(_off(k), CHUNK)],
                              didx[p].at[0], isem[p]).wait()

    idx_issue(0, 0)
    idx_issue(1, 1)

    def pair(j, _):
        for p in (0, 1):
            r = j * 2 + p
            idx_wait(p, r)
            pltpu.sync_copy(ones, acc.at[didx[p].at[0]], add=True)

            @pl.when(j < WCHUNKS // 2 - 1)
            def _():
                idx_issue(p, r + 2)
        return 0

    lax.fori_loop(0, WCHUNKS // 2, pair, 0)
    plsc.subcore_barrier()

    @pl.when(c == 0)
    def _():
        _sliced_copy2(acc, out0, base, s)

    @pl.when(c == 1)
    def _():
        _sliced_copy2(acc, out1, base, s)


_sc_counts = pl.kernel(
    _cnt_body,
    out_type=(jax.ShapeDtypeStruct((NP, F), jnp.float32),
              jax.ShapeDtypeStruct((NP, F), jnp.float32)),
    mesh=_SC_MESH,
    scratch_types=[
        pltpu.VMEM_SHARED((NP, F), jnp.float32),
        pltpu.VMEM((1, CHUNK), jnp.int32),
        pltpu.VMEM((1, CHUNK), jnp.int32),
        pltpu.VMEM((CHUNK, F), jnp.float32),
        pltpu.SemaphoreType.DMA,
        pltpu.SemaphoreType.DMA,
    ],
)


BLK = 2000


def _mm2(xb, wl, wr, bl):
    y = lax.dot_general(xb, wl, (((1,), (1,)), ((), ())),
                        preferred_element_type=jnp.float32)
    z = lax.dot_general(xb, wr, (((1,), (1,)), ((), ())),
                        preferred_element_type=jnp.float32) + bl[0][None, :]
    return y, z


def _tc_first_body(x_ref, wl_ref, wr_ref, bl_ref, y_ref, z_ref):
    y, z = _mm2(x_ref[...], wl_ref[...], wr_ref[...], bl_ref)
    y_ref[...] = y
    z_ref[...] = z


def _tc_first(x, wl, wr, bl):
    return pl.pallas_call(
        _tc_first_body,
        grid=(N // BLK,),
        in_specs=[
            pl.BlockSpec((BLK, F), lambda i: (i, 0)),
            pl.BlockSpec((F, F), lambda i: (0, 0)),
            pl.BlockSpec((F, F), lambda i: (0, 0)),
            pl.BlockSpec((1, F), lambda i: (0, 0)),
        ],
        out_specs=[
            pl.BlockSpec((BLK, F), lambda i: (i, 0)),
            pl.BlockSpec((BLK, F), lambda i: (i, 0)),
        ],
        out_shape=[jax.ShapeDtypeStruct((N, F), jnp.float32),
                   jax.ShapeDtypeStruct((N, F), jnp.float32)],
    )(x, wl, wr, bl.reshape(1, F))


def _combine(x_ref, p0_ref, p1_ref, c0_ref, c1_ref, zp_ref):
    cnt = jnp.maximum(c0_ref[...] + c1_ref[...], 1.0)
    mean = (p0_ref[...] + p1_ref[...]) / cnt
    return jnp.maximum(mean + zp_ref[...], 0.0) + x_ref[...]


def _tc_mid_body(x_ref, p0_ref, p1_ref, c0_ref, c1_ref, zp_ref, wl_ref,
                 wr_ref, bl_ref, xn_ref, y_ref, z_ref):
    xn = _combine(x_ref, p0_ref, p1_ref, c0_ref, c1_ref, zp_ref)
    xn_ref[...] = xn
    y, z = _mm2(xn, wl_ref[...], wr_ref[...], bl_ref)
    y_ref[...] = y
    z_ref[...] = z


def _tc_mid(x, p0, p1, c0, c1, zp, wl, wr, bl):
    blkspec = pl.BlockSpec((BLK, F), lambda i: (i, 0))
    wspec = pl.BlockSpec((F, F), lambda i: (0, 0))
    return pl.pallas_call(
        _tc_mid_body,
        grid=(N // BLK,),
        in_specs=[blkspec] * 6 + [wspec, wspec,
                  pl.BlockSpec((1, F), lambda i: (0, 0))],
        out_specs=[blkspec, blkspec, blkspec],
        out_shape=[jax.ShapeDtypeStruct((N, F), jnp.float32)] * 3,
    )(x, p0, p1, c0, c1, zp, wl, wr, bl.reshape(1, F))


def _tc_last_body(x_ref, p0_ref, p1_ref, c0_ref, c1_ref, zp_ref, xn_ref):
    xn_ref[...] = _combine(x_ref, p0_ref, p1_ref, c0_ref, c1_ref, zp_ref)


def _tc_last(x, p0, p1, c0, c1, zp):
    blkspec = pl.BlockSpec((BLK, F), lambda i: (i, 0))
    return pl.pallas_call(
        _tc_last_body,
        grid=(N // BLK,),
        in_specs=[blkspec] * 6,
        out_specs=blkspec,
        out_shape=jax.ShapeDtypeStruct((N, F), jnp.float32),
    )(x, p0, p1, c0, c1, zp)


SBLK = 1000
GPAD = 104


def _softmax_body(t_ref, x_ref, b_ref, out_ref, m_ref, den_ref, num_ref):
    p = pl.program_id(0)
    blk = pl.program_id(1)
    t = t_ref[0, 0]
    xb = x_ref[...]
    bb = b_ref[...]
    g_lo = jnp.min(bb)
    g_hi = jnp.max(bb)

    @pl.when(jnp.logical_and(p == 0, blk == 0))
    def _():
        m_ref[...] = jnp.full((GPAD, F), -3.0e38, jnp.float32)
        den_ref[...] = jnp.zeros((GPAD, F), jnp.float32)
        num_ref[...] = jnp.zeros((GPAD, F), jnp.float32)

    logits = xb * t

    @pl.when(p == 0)
    def _():
        def body(g, _):
            sel = jnp.where(bb == g, logits, -3.0e38)
            m = jnp.max(sel, axis=0, keepdims=True)
            m_ref[pl.ds(g, 1), :] = jnp.maximum(m_ref[pl.ds(g, 1), :], m)
            return 0

        lax.fori_loop(g_lo, g_hi + 1, body, 0)

    @pl.when(p == 1)
    def _():
        def body(g, _):
            mask = bb == g
            mg = m_ref[pl.ds(g, 1), :]
            e = jnp.where(mask, jnp.exp(logits - mg), 0.0)
            den_ref[pl.ds(g, 1), :] += jnp.sum(e, axis=0, keepdims=True)
            num_ref[pl.ds(g, 1), :] += jnp.sum(xb * e, axis=0, keepdims=True)
            return 0

        lax.fori_loop(g_lo, g_hi + 1, body, 0)

    @pl.when(jnp.logical_and(p == 1, blk == pl.num_programs(1) - 1))
    def _():
        den = den_ref[...]
        out_ref[...] = jnp.where(den > 0.0, num_ref[...] / den, 0.0)


def _tc_softmax(x, batchb, t):
    return pl.pallas_call(
        _softmax_body,
        grid=(2, N // SBLK),
        in_specs=[
            pl.BlockSpec(memory_space=pltpu.SMEM),
            pl.BlockSpec((SBLK, F), lambda p, b: (b, 0)),
            pl.BlockSpec((SBLK, F), lambda p, b: (b, 0)),
        ],
        out_specs=pl.BlockSpec((GPAD, F), lambda p, b: (0, 0)),
        out_shape=jax.ShapeDtypeStruct((GPAD, F), jnp.float32),
        scratch_shapes=[
            pltpu.VMEM((GPAD, F), jnp.float32),
            pltpu.VMEM((GPAD, F), jnp.float32),
            pltpu.VMEM((GPAD, F), jnp.float32),
        ],
        compiler_params=pltpu.CompilerParams(
            dimension_semantics=("arbitrary", "arbitrary")),
    )(t.reshape(1, 1), x, batchb)


def _pad_edges(src, dst):
    npad = EPAD - E
    pad_i = jnp.arange(npad, dtype=jnp.int32)
    src_p = jnp.concatenate([src, (pad_i * 37) % N])
    dst_p = jnp.concatenate([dst, N + (pad_i % NROWS_PAD)])
    return src_p, dst_p


def kernel(x, edge_index, batch,
           Wl0, bl0, Wr0, Wl1, bl1, Wr1, Wl2, bl2, Wr2,
           Wl3, bl3, Wr3, Wl4, bl4, Wr4, Wl5, bl5, Wr5, t):
    src_p, dst_p = _pad_edges(edge_index[0], edge_index[1])

    c0, c1 = _sc_counts(dst_p)
    batchb = jnp.broadcast_to(batch[:, None], (N, F))

    params = [(Wl0, bl0, Wr0), (Wl1, bl1, Wr1), (Wl2, bl2, Wr2),
              (Wl3, bl3, Wr3), (Wl4, bl4, Wr4), (Wl5, bl5, Wr5)]

    y, z = _tc_first(x, Wl0, Wr0, bl0)
    for i in range(6):
        p0, p1 = _sc_aggregate(y, src_p, dst_p)
        if i < 5:
            wl, bl, wr = params[i + 1]
            x, y, z = _tc_mid(x, p0, p1, c0, c1, z, wl, wr, bl)
        else:
            x = _tc_last(x, p0, p1, c0, c1, z)

    return _tc_softmax(x, batchb, t)[:G]
